# baseline (speedup 1.0000x reference)
import functools

import jax
import jax.numpy as jnp
from jax import lax
from jax.experimental import pallas as pl
from jax.experimental.pallas import tpu as pltpu
from jax.experimental.pallas import tpu_sc as plsc

N_NODES = 100000
IN_DIM = 60
BATCH = 16384
FANOUT = 6
DIMS = 20
TD = 24
HD = 128

NC, NS = 2, 16
NW = NC * NS
BPW = BATCH // NW
CH = 64
NCHUNK = BPW // CH
IDX_TILE = 128


def _table_body(x_ref, w1a_ref, w1b_ref, f1_ref, f2_ref):
    x = x_ref[...]
    f1_ref[...] = jnp.dot(x, w1a_ref[...], preferred_element_type=jnp.float32)
    f2_ref[...] = jnp.dot(x, w1b_ref[...], preferred_element_type=jnp.float32)


def _make_tables(features, W1):
    rows = features.shape[0]
    blk = 8192
    grid = (rows + blk - 1) // blk
    return pl.pallas_call(
        _table_body,
        grid=(grid,),
        in_specs=[
            pl.BlockSpec((blk, IN_DIM), lambda i: (i, 0)),
            pl.BlockSpec((IN_DIM, TD), lambda i: (0, 0)),
            pl.BlockSpec((IN_DIM, TD), lambda i: (0, 0)),
        ],
        out_specs=[
            pl.BlockSpec((blk, TD), lambda i: (i, 0)),
            pl.BlockSpec((blk, TD), lambda i: (i, 0)),
        ],
        out_shape=[
            jax.ShapeDtypeStruct((rows, TD), jnp.float32),
            jax.ShapeDtypeStruct((rows, TD), jnp.float32),
        ],
    )(features,
      jnp.pad(W1[:IN_DIM], ((0, 0), (0, TD - DIMS))),
      jnp.pad(W1[IN_DIM:], ((0, 0), (0, TD - DIMS))))


def _head_body(h_ref, w2_ref, b2_ref, o_ref):
    h = h_ref[...][:, :2 * DIMS]
    acc = jnp.dot(h, w2_ref[...], preferred_element_type=jnp.float32)
    o_ref[...] = jnp.maximum(acc + b2_ref[...], 0.0)


def _head(H, W2, b2):
    blk = 2048
    return pl.pallas_call(
        _head_body,
        grid=(BATCH // blk,),
        in_specs=[
            pl.BlockSpec((blk, HD), lambda i: (i, 0)),

            pl.BlockSpec((2 * DIMS, DIMS), lambda i: (0, 0)),
            pl.BlockSpec((1, DIMS), lambda i: (0, 0)),
        ],
        out_specs=pl.BlockSpec((blk, DIMS), lambda i: (i, 0)),
        out_shape=jax.ShapeDtypeStruct((BATCH, DIMS), jnp.float32),
    )(H, W2, b2.reshape(1, DIMS))


def _gather_body(f1_hbm, f2_hbm, idx_hbm, n1_hbm, n2_hbm, b1_hbm, out_hbm,
                 idxv, n1p, n2p, n1v, n2v, rs, rn1a, rn1b, rn2, hb, b1v, sem):
    wid = lax.axis_index("s") * NC + lax.axis_index("c")
    base = wid * BPW
    pltpu.sync_copy(b1_hbm, b1v)
    b1A = b1v[pl.ds(0, 16)]
    b1B = b1v[pl.ds(16, 16)]
    sixth = jnp.float32(1.0 / FANOUT)

    @pl.loop(0, NCHUNK)
    def chunk(ci):
        cb = base + ci * CH
        pltpu.sync_copy(idx_hbm.at[pl.ds(cb, CH)], idxv)
        pltpu.sync_copy(n1_hbm.at[pl.ds(cb, CH)], n1p)
        pltpu.sync_copy(n2_hbm.at[pl.ds(cb, CH)], n2p)

        @pl.loop(0, CH * FANOUT // 16)
        def c1(g):
            p = g * 16 + lax.iota(jnp.int32, 16)
            r = lax.shift_right_arithmetic(p * 43691, 18)
            c = p - r * FANOUT
            n1v[pl.ds(g * 16, 16)] = plsc.load_gather(n1p, [r, c])

        @pl.loop(0, CH * FANOUT * FANOUT // 16)
        def c2(g):
            p = g * 16 + lax.iota(jnp.int32, 16)
            r = lax.shift_right_arithmetic(p * 29128, 20)
            rem = p - r * (FANOUT * FANOUT)
            m = lax.shift_right_arithmetic(rem * 43691, 18)
            c = rem - m * FANOUT
            n2v[pl.ds(g * 16, 16)] = plsc.load_gather(n2p, [r, m, c])

        cps = [pltpu.async_copy(f1_hbm.at[idxv], rs, sem)]
        for k in range(CH * FANOUT // IDX_TILE):
            src = pl.ds(k * IDX_TILE, IDX_TILE)
            dst = pl.ds(k * IDX_TILE, IDX_TILE)
            cps.append(pltpu.async_copy(f1_hbm.at[n1v.at[src]], rn1a.at[dst], sem))
            cps.append(pltpu.async_copy(f2_hbm.at[n1v.at[src]], rn1b.at[dst], sem))
        for k in range(CH * FANOUT * FANOUT // IDX_TILE):
            src = pl.ds(k * IDX_TILE, IDX_TILE)
            dst = pl.ds(k * IDX_TILE, IDX_TILE)
            cps.append(pltpu.async_copy(f2_hbm.at[n2v.at[src]], rn2.at[dst], sem))
        for cp in cps:
            cp.wait()

        @pl.loop(0, CH)
        def elem(e):
            zero = jnp.zeros((16,), jnp.float32)
            acc0 = zero
            acc1 = zero
            sb0 = zero
            sb1 = zero
            for i in range(FANOUT):
                g = e * FANOUT + i
                s0 = zero
                s1 = zero
                for j in range(FANOUT):
                    r = g * FANOUT + j
                    s0 = s0 + rn2[r, pl.ds(0, 16)]
                    s1 = s1 + rn2[r, pl.ds(4, 16)]
                q0 = jnp.maximum(rn1a[g, pl.ds(0, 16)] + sixth * s0 + b1A, 0.0)
                q1 = jnp.maximum(rn1a[g, pl.ds(4, 16)] + sixth * s1 + b1B, 0.0)
                acc0 = acc0 + q0
                acc1 = acc1 + q1
                sb0 = sb0 + rn1b[g, pl.ds(0, 16)]
                sb1 = sb1 + rn1b[g, pl.ds(4, 16)]
            hs0 = jnp.maximum(rs[e, pl.ds(0, 16)] + sixth * sb0 + b1A, 0.0)
            hs1 = jnp.maximum(rs[e, pl.ds(4, 16)] + sixth * sb1 + b1B, 0.0)
            hb[e, pl.ds(0, 16)] = hs0
            hb[e, pl.ds(4, 16)] = hs1
            hb[e, pl.ds(20, 16)] = sixth * acc0
            hb[e, pl.ds(24, 16)] = sixth * acc1

        pltpu.sync_copy(hb, out_hbm.at[pl.ds(cb, CH)])


def _gather_kernel(F1, F2, idx, n1m, n2m, b1cat):
    mesh = plsc.VectorSubcoreMesh(core_axis_name="c", subcore_axis_name="s")
    run = functools.partial(
        pl.kernel,
        out_type=jax.ShapeDtypeStruct((BATCH, HD), jnp.float32),
        mesh=mesh,
        compiler_params=pltpu.CompilerParams(use_tc_tiling_on_sc=False, needs_layout_passes=False),
        scratch_types=[
            pltpu.VMEM((CH,), jnp.int32),
            pltpu.VMEM((CH, FANOUT), jnp.int32),
            pltpu.VMEM((CH, FANOUT, FANOUT), jnp.int32),
            pltpu.VMEM((CH * FANOUT,), jnp.int32),
            pltpu.VMEM((CH * FANOUT * FANOUT,), jnp.int32),
            pltpu.VMEM((CH, TD), jnp.float32),
            pltpu.VMEM((CH * FANOUT, TD), jnp.float32),
            pltpu.VMEM((CH * FANOUT, TD), jnp.float32),
            pltpu.VMEM((CH * FANOUT * FANOUT, TD), jnp.float32),
            pltpu.VMEM((CH, HD), jnp.float32),
            pltpu.VMEM((32,), jnp.float32),
            pltpu.SemaphoreType.DMA,
        ],
    )(_gather_body)
    return run(F1, F2, idx, n1m, n2m, b1cat)


def kernel(features, idx, first_order_neighs, second_order_neighs,
           W1, b1, W2, b2):
    F1, F2 = _make_tables(features, W1)
    n1m = first_order_neighs
    n2m = second_order_neighs
    b1cat = jnp.concatenate([b1[0:16], b1[4:20]])
    H = _gather_kernel(F1, F2, idx, n1m, n2m, b1cat)
    return _head(H, W2, b2)


# speedup vs baseline: 1.1573x; 1.1573x over previous
import functools

import jax
import jax.numpy as jnp
from jax import lax
from jax.experimental import pallas as pl
from jax.experimental.pallas import tpu as pltpu
from jax.experimental.pallas import tpu_sc as plsc

N_NODES = 100000
IN_DIM = 60
BATCH = 16384
FANOUT = 6
DIMS = 20
TD = 24
HD = 128

NC, NS = 2, 16
NW = NC * NS
BPW = BATCH // NW
CH = 64
NCHUNK = BPW // CH
IDX_TILE = 128


def _table_body(x_ref, w1a_ref, w1b_ref, f1_ref, f2_ref):
    x = x_ref[...]
    f1_ref[...] = jnp.dot(x, w1a_ref[...], preferred_element_type=jnp.float32)
    f2_ref[...] = jnp.dot(x, w1b_ref[...], preferred_element_type=jnp.float32)


def _make_tables(features, W1):
    rows = features.shape[0]
    blk = 8192
    grid = (rows + blk - 1) // blk
    return pl.pallas_call(
        _table_body,
        grid=(grid,),
        in_specs=[
            pl.BlockSpec((blk, IN_DIM), lambda i: (i, 0)),
            pl.BlockSpec((IN_DIM, TD), lambda i: (0, 0)),
            pl.BlockSpec((IN_DIM, TD), lambda i: (0, 0)),
        ],
        out_specs=[
            pl.BlockSpec((blk, TD), lambda i: (i, 0)),
            pl.BlockSpec((blk, TD), lambda i: (i, 0)),
        ],
        out_shape=[
            jax.ShapeDtypeStruct((rows, TD), jnp.float32),
            jax.ShapeDtypeStruct((rows, TD), jnp.float32),
        ],
    )(features,
      jnp.pad(W1[:IN_DIM], ((0, 0), (0, TD - DIMS))),
      jnp.pad(W1[IN_DIM:], ((0, 0), (0, TD - DIMS))))


def _head_body(h_ref, w2_ref, b2_ref, o_ref):
    h = h_ref[...][:, :2 * DIMS]
    acc = jnp.dot(h, w2_ref[...], preferred_element_type=jnp.float32)
    o_ref[...] = jnp.maximum(acc + b2_ref[...], 0.0)


def _head(H, W2, b2):
    blk = 2048
    return pl.pallas_call(
        _head_body,
        grid=(BATCH // blk,),
        in_specs=[
            pl.BlockSpec((blk, HD), lambda i: (i, 0)),

            pl.BlockSpec((2 * DIMS, DIMS), lambda i: (0, 0)),
            pl.BlockSpec((1, DIMS), lambda i: (0, 0)),
        ],
        out_specs=pl.BlockSpec((blk, DIMS), lambda i: (i, 0)),
        out_shape=jax.ShapeDtypeStruct((BATCH, DIMS), jnp.float32),
    )(H, W2, b2.reshape(1, DIMS))


N1_OFF = BATCH
N2_OFF = BATCH + BATCH * FANOUT


def _gather_body(f1_hbm, f2_hbm, cidx_hbm, b1_hbm, out_hbm,
                 idxv, n1v, n2v, rs, rn1a, rn1b, rn2, hb, b1v, sem):
    wid = lax.axis_index("s") * NC + lax.axis_index("c")
    base = wid * BPW
    pltpu.sync_copy(b1_hbm, b1v)
    pltpu.sync_copy(cidx_hbm.at[pl.ds(base, BPW)], idxv)
    pltpu.sync_copy(cidx_hbm.at[pl.ds(N1_OFF + wid * BPW * FANOUT,
                                      BPW * FANOUT)], n1v)
    pltpu.sync_copy(cidx_hbm.at[pl.ds(N2_OFF + wid * BPW * FANOUT * FANOUT,
                                      BPW * FANOUT * FANOUT)], n2v)
    b1A = b1v[pl.ds(0, 16)]
    b1B = b1v[pl.ds(16, 16)]
    sixth = jnp.float32(1.0 / FANOUT)

    @pl.loop(0, NCHUNK)
    def chunk(ci):
        cb = base + ci * CH
        cps = [pltpu.async_copy(f1_hbm.at[idxv.at[pl.ds(ci * CH, CH)]], rs, sem)]
        for k in range(CH * FANOUT // IDX_TILE):
            s = pl.ds(ci * CH * FANOUT + k * IDX_TILE, IDX_TILE)
            dst = pl.ds(k * IDX_TILE, IDX_TILE)
            cps.append(pltpu.async_copy(f1_hbm.at[n1v.at[s]], rn1a.at[dst], sem))
            cps.append(pltpu.async_copy(f2_hbm.at[n1v.at[s]], rn1b.at[dst], sem))
        for k in range(CH * FANOUT * FANOUT // IDX_TILE):
            s = pl.ds(ci * CH * FANOUT * FANOUT + k * IDX_TILE, IDX_TILE)
            dst = pl.ds(k * IDX_TILE, IDX_TILE)
            cps.append(pltpu.async_copy(f2_hbm.at[n2v.at[s]], rn2.at[dst], sem))
        for cp in cps:
            cp.wait()

        @pl.loop(0, CH)
        def elem(e):
            zero = jnp.zeros((16,), jnp.float32)
            acc0 = zero
            acc1 = zero
            sb0 = zero
            sb1 = zero
            for i in range(FANOUT):
                g = e * FANOUT + i
                s0 = zero
                s1 = zero
                for j in range(FANOUT):
                    r = g * FANOUT + j
                    s0 = s0 + rn2[r, pl.ds(0, 16)]
                    s1 = s1 + rn2[r, pl.ds(4, 16)]
                q0 = jnp.maximum(rn1a[g, pl.ds(0, 16)] + sixth * s0 + b1A, 0.0)
                q1 = jnp.maximum(rn1a[g, pl.ds(4, 16)] + sixth * s1 + b1B, 0.0)
                acc0 = acc0 + q0
                acc1 = acc1 + q1
                sb0 = sb0 + rn1b[g, pl.ds(0, 16)]
                sb1 = sb1 + rn1b[g, pl.ds(4, 16)]
            hs0 = jnp.maximum(rs[e, pl.ds(0, 16)] + sixth * sb0 + b1A, 0.0)
            hs1 = jnp.maximum(rs[e, pl.ds(4, 16)] + sixth * sb1 + b1B, 0.0)
            hb[e, pl.ds(0, 16)] = hs0
            hb[e, pl.ds(4, 16)] = hs1
            hb[e, pl.ds(20, 16)] = sixth * acc0
            hb[e, pl.ds(24, 16)] = sixth * acc1

        pltpu.sync_copy(hb, out_hbm.at[pl.ds(cb, CH)])


def _gather_kernel(F1, F2, cidx, b1cat):
    mesh = plsc.VectorSubcoreMesh(core_axis_name="c", subcore_axis_name="s")
    run = functools.partial(
        pl.kernel,
        out_type=jax.ShapeDtypeStruct((BATCH, HD), jnp.float32),
        mesh=mesh,
        compiler_params=pltpu.CompilerParams(use_tc_tiling_on_sc=False, needs_layout_passes=False),
        scratch_types=[
            pltpu.VMEM((BPW,), jnp.int32),
            pltpu.VMEM((BPW * FANOUT,), jnp.int32),
            pltpu.VMEM((BPW * FANOUT * FANOUT,), jnp.int32),
            pltpu.VMEM((CH, TD), jnp.float32),
            pltpu.VMEM((CH * FANOUT, TD), jnp.float32),
            pltpu.VMEM((CH * FANOUT, TD), jnp.float32),
            pltpu.VMEM((CH * FANOUT * FANOUT, TD), jnp.float32),
            pltpu.VMEM((CH, HD), jnp.float32),
            pltpu.VMEM((32,), jnp.float32),
            pltpu.SemaphoreType.DMA,
        ],
    )(_gather_body)
    return run(F1, F2, cidx, b1cat)


def kernel(features, idx, first_order_neighs, second_order_neighs,
           W1, b1, W2, b2):
    F1, F2 = _make_tables(features, W1)
    cidx = jnp.concatenate([idx, first_order_neighs.reshape(-1),
                            second_order_neighs.reshape(-1)])
    b1cat = jnp.concatenate([b1[0:16], b1[4:20]])
    H = _gather_kernel(F1, F2, cidx, b1cat)
    return _head(H, W2, b2)


# speedup vs baseline: 1.1603x; 1.0026x over previous
import functools

import jax
import jax.numpy as jnp
from jax import lax
from jax.experimental import pallas as pl
from jax.experimental.pallas import tpu as pltpu
from jax.experimental.pallas import tpu_sc as plsc

N_NODES = 100000
IN_DIM = 60
BATCH = 16384
FANOUT = 6
DIMS = 20
TD = 24
HD = 128

NC, NS = 2, 16
NW = NC * NS
BPW = BATCH // NW
CH = 64
NCHUNK = BPW // CH
IDX_TILE = 128


def _table_body(x_ref, w1a_ref, w1b_ref, b1_ref, f1_ref, f2_ref):
    x = x_ref[...]
    f1_ref[...] = (jnp.dot(x, w1a_ref[...], preferred_element_type=jnp.float32)
                   + b1_ref[...])
    f2_ref[...] = jnp.dot(x, w1b_ref[...], preferred_element_type=jnp.float32)


def _make_tables(features, W1, b1):
    rows = features.shape[0]
    blk = 8192
    grid = (rows + blk - 1) // blk
    return pl.pallas_call(
        _table_body,
        grid=(grid,),
        in_specs=[
            pl.BlockSpec((blk, IN_DIM), lambda i: (i, 0)),
            pl.BlockSpec((IN_DIM, TD), lambda i: (0, 0)),
            pl.BlockSpec((IN_DIM, TD), lambda i: (0, 0)),
            pl.BlockSpec((1, TD), lambda i: (0, 0)),
        ],
        out_specs=[
            pl.BlockSpec((blk, TD), lambda i: (i, 0)),
            pl.BlockSpec((blk, TD), lambda i: (i, 0)),
        ],
        out_shape=[
            jax.ShapeDtypeStruct((rows, TD), jnp.float32),
            jax.ShapeDtypeStruct((rows, TD), jnp.float32),
        ],
    )(features,
      jnp.pad(W1[:IN_DIM], ((0, 0), (0, TD - DIMS))),
      jnp.pad(W1[IN_DIM:], ((0, 0), (0, TD - DIMS))),
      jnp.pad(b1, (0, TD - DIMS)).reshape(1, TD))


def _head_body(h_ref, w2_ref, b2_ref, o_ref):
    h = h_ref[...][:, :2 * DIMS]
    acc = jnp.dot(h, w2_ref[...], preferred_element_type=jnp.float32)
    o_ref[...] = jnp.maximum(acc + b2_ref[...], 0.0)


def _head(H, W2, b2):
    blk = 2048
    return pl.pallas_call(
        _head_body,
        grid=(BATCH // blk,),
        in_specs=[
            pl.BlockSpec((blk, HD), lambda i: (i, 0)),

            pl.BlockSpec((2 * DIMS, DIMS), lambda i: (0, 0)),
            pl.BlockSpec((1, DIMS), lambda i: (0, 0)),
        ],
        out_specs=pl.BlockSpec((blk, DIMS), lambda i: (i, 0)),
        out_shape=jax.ShapeDtypeStruct((BATCH, DIMS), jnp.float32),
    )(H, W2, b2.reshape(1, DIMS))


N1_OFF = BATCH
N2_OFF = BATCH + BATCH * FANOUT


def _gather_body(f1_hbm, f2_hbm, idx_hbm, n1_hbm, n2_hbm, out_hbm,
                 idxv, n1v, n2v, rs, rn1a, rn1b, rn2, hb, sem):
    wid = lax.axis_index("s") * NC + lax.axis_index("c")
    base = wid * BPW
    pltpu.sync_copy(idx_hbm.at[pl.ds(base, BPW)], idxv)
    pltpu.sync_copy(n1_hbm.at[pl.ds(base * FANOUT, BPW * FANOUT)], n1v)
    pltpu.sync_copy(n2_hbm.at[pl.ds(base * FANOUT * FANOUT,
                                    BPW * FANOUT * FANOUT)], n2v)
    sixth = jnp.float32(1.0 / FANOUT)

    @pl.loop(0, NCHUNK)
    def chunk(ci):
        cb = base + ci * CH
        cps = [pltpu.async_copy(f1_hbm.at[idxv.at[pl.ds(ci * CH, CH)]], rs, sem)]
        for k in range(CH * FANOUT // IDX_TILE):
            s = pl.ds(ci * CH * FANOUT + k * IDX_TILE, IDX_TILE)
            dst = pl.ds(k * IDX_TILE, IDX_TILE)
            cps.append(pltpu.async_copy(f1_hbm.at[n1v.at[s]], rn1a.at[dst], sem))
            cps.append(pltpu.async_copy(f2_hbm.at[n1v.at[s]], rn1b.at[dst], sem))
        for k in range(CH * FANOUT * FANOUT // IDX_TILE):
            s = pl.ds(ci * CH * FANOUT * FANOUT + k * IDX_TILE, IDX_TILE)
            dst = pl.ds(k * IDX_TILE, IDX_TILE)
            cps.append(pltpu.async_copy(f2_hbm.at[n2v.at[s]], rn2.at[dst], sem))
        for cp in cps:
            cp.wait()

        @pl.loop(0, CH)
        def elem(e):
            zero = jnp.zeros((16,), jnp.float32)
            acc0 = zero
            acc1 = zero
            sb0 = zero
            sb1 = zero
            for i in range(FANOUT):
                g = e * FANOUT + i
                s0 = zero
                s1 = zero
                for j in range(FANOUT):
                    r = g * FANOUT + j
                    s0 = s0 + rn2[r, pl.ds(0, 16)]
                    s1 = s1 + rn2[r, pl.ds(4, 16)]
                q0 = jnp.maximum(rn1a[g, pl.ds(0, 16)] + sixth * s0, 0.0)
                q1 = jnp.maximum(rn1a[g, pl.ds(4, 16)] + sixth * s1, 0.0)
                acc0 = acc0 + q0
                acc1 = acc1 + q1
                sb0 = sb0 + rn1b[g, pl.ds(0, 16)]
                sb1 = sb1 + rn1b[g, pl.ds(4, 16)]
            hs0 = jnp.maximum(rs[e, pl.ds(0, 16)] + sixth * sb0, 0.0)
            hs1 = jnp.maximum(rs[e, pl.ds(4, 16)] + sixth * sb1, 0.0)
            hb[e, pl.ds(0, 16)] = hs0
            hb[e, pl.ds(4, 16)] = hs1
            hb[e, pl.ds(20, 16)] = sixth * acc0
            hb[e, pl.ds(24, 16)] = sixth * acc1

        pltpu.sync_copy(hb, out_hbm.at[pl.ds(cb, CH)])


def _gather_kernel(F1, F2, idx, n1f, n2f):
    mesh = plsc.VectorSubcoreMesh(core_axis_name="c", subcore_axis_name="s")
    run = functools.partial(
        pl.kernel,
        out_type=jax.ShapeDtypeStruct((BATCH, HD), jnp.float32),
        mesh=mesh,
        compiler_params=pltpu.CompilerParams(use_tc_tiling_on_sc=False, needs_layout_passes=False),
        scratch_types=[
            pltpu.VMEM((BPW,), jnp.int32),
            pltpu.VMEM((BPW * FANOUT,), jnp.int32),
            pltpu.VMEM((BPW * FANOUT * FANOUT,), jnp.int32),
            pltpu.VMEM((CH, TD), jnp.float32),
            pltpu.VMEM((CH * FANOUT, TD), jnp.float32),
            pltpu.VMEM((CH * FANOUT, TD), jnp.float32),
            pltpu.VMEM((CH * FANOUT * FANOUT, TD), jnp.float32),
            pltpu.VMEM((CH, HD), jnp.float32),
            pltpu.SemaphoreType.DMA,
        ],
    )(_gather_body)
    return run(F1, F2, idx, n1f, n2f)


def kernel(features, idx, first_order_neighs, second_order_neighs,
           W1, b1, W2, b2):
    F1, F2 = _make_tables(features, W1, b1)
    H = _gather_kernel(F1, F2, idx, first_order_neighs.reshape(-1),
                       second_order_neighs.reshape(-1))
    return _head(H, W2, b2)


# speedup vs baseline: 1.2524x; 1.0794x over previous
import functools

import jax
import jax.numpy as jnp
from jax import lax
from jax.experimental import pallas as pl
from jax.experimental.pallas import tpu as pltpu
from jax.experimental.pallas import tpu_sc as plsc

N_NODES = 100000
IN_DIM = 60
BATCH = 16384
FANOUT = 6
DIMS = 20
TD = 24
HD = 128

NC, NS = 2, 16
NW = NC * NS
BPW = BATCH // NW
CH = 32
NCHUNK = BPW // CH
IDX_TILE = 128


def _table_body(x_ref, w1a_ref, w1b_ref, b1_ref, f1_ref, f2_ref):
    x = x_ref[...]
    f1_ref[...] = (jnp.dot(x, w1a_ref[...], preferred_element_type=jnp.float32)
                   + b1_ref[...])
    f2_ref[...] = jnp.dot(x, w1b_ref[...], preferred_element_type=jnp.float32)


def _make_tables(features, W1, b1):
    rows = features.shape[0]
    blk = 8192
    grid = (rows + blk - 1) // blk
    return pl.pallas_call(
        _table_body,
        grid=(grid,),
        in_specs=[
            pl.BlockSpec((blk, IN_DIM), lambda i: (i, 0)),
            pl.BlockSpec((IN_DIM, TD), lambda i: (0, 0)),
            pl.BlockSpec((IN_DIM, TD), lambda i: (0, 0)),
            pl.BlockSpec((1, TD), lambda i: (0, 0)),
        ],
        out_specs=[
            pl.BlockSpec((blk, TD), lambda i: (i, 0)),
            pl.BlockSpec((blk, TD), lambda i: (i, 0)),
        ],
        out_shape=[
            jax.ShapeDtypeStruct((rows, TD), jnp.float32),
            jax.ShapeDtypeStruct((rows, TD), jnp.float32),
        ],
    )(features,
      jnp.pad(W1[:IN_DIM], ((0, 0), (0, TD - DIMS))),
      jnp.pad(W1[IN_DIM:], ((0, 0), (0, TD - DIMS))),
      jnp.pad(b1, (0, TD - DIMS)).reshape(1, TD))


def _head_body(h_ref, w2_ref, b2_ref, o_ref):
    h = h_ref[...][:, :2 * DIMS]
    acc = jnp.dot(h, w2_ref[...], preferred_element_type=jnp.float32)
    o_ref[...] = jnp.maximum(acc + b2_ref[...], 0.0)


def _head(H, W2, b2):
    blk = 2048
    return pl.pallas_call(
        _head_body,
        grid=(BATCH // blk,),
        in_specs=[
            pl.BlockSpec((blk, HD), lambda i: (i, 0)),

            pl.BlockSpec((2 * DIMS, DIMS), lambda i: (0, 0)),
            pl.BlockSpec((1, DIMS), lambda i: (0, 0)),
        ],
        out_specs=pl.BlockSpec((blk, DIMS), lambda i: (i, 0)),
        out_shape=jax.ShapeDtypeStruct((BATCH, DIMS), jnp.float32),
    )(H, W2, b2.reshape(1, DIMS))


N1_OFF = BATCH
N2_OFF = BATCH + BATCH * FANOUT


def _fire(ci, f1_hbm, f2_hbm, idxv, n1v, n2v, rs, rn1a, rn1b, rn2, sem):
    pltpu.async_copy(f1_hbm.at[idxv.at[pl.ds(ci * CH, CH)]], rs, sem)
    n1o = ci * CH * FANOUT
    pltpu.async_copy(f1_hbm.at[n1v.at[pl.ds(n1o, 128)]],
                     rn1a.at[pl.ds(0, 128)], sem)
    pltpu.async_copy(f1_hbm.at[n1v.at[pl.ds(n1o + 128, 64)]],
                     rn1a.at[pl.ds(128, 64)], sem)
    pltpu.async_copy(f2_hbm.at[n1v.at[pl.ds(n1o, 128)]],
                     rn1b.at[pl.ds(0, 128)], sem)
    pltpu.async_copy(f2_hbm.at[n1v.at[pl.ds(n1o + 128, 64)]],
                     rn1b.at[pl.ds(128, 64)], sem)
    n2o = ci * CH * FANOUT * FANOUT
    for k in range(CH * FANOUT * FANOUT // IDX_TILE):
        pltpu.async_copy(f2_hbm.at[n2v.at[pl.ds(n2o + k * IDX_TILE, IDX_TILE)]],
                         rn2.at[pl.ds(k * IDX_TILE, IDX_TILE)], sem)


def _drain(f1_hbm, f2_hbm, idxv, n1v, n2v, rs, rn1a, rn1b, rn2, sem):
    pltpu.make_async_copy(f1_hbm.at[idxv.at[pl.ds(0, CH)]], rs, sem).wait()
    pltpu.make_async_copy(f1_hbm.at[n1v.at[pl.ds(0, 128)]],
                          rn1a.at[pl.ds(0, 128)], sem).wait()
    pltpu.make_async_copy(f1_hbm.at[n1v.at[pl.ds(0, 64)]],
                          rn1a.at[pl.ds(128, 64)], sem).wait()
    pltpu.make_async_copy(f2_hbm.at[n1v.at[pl.ds(0, 128)]],
                          rn1b.at[pl.ds(0, 128)], sem).wait()
    pltpu.make_async_copy(f2_hbm.at[n1v.at[pl.ds(0, 64)]],
                          rn1b.at[pl.ds(128, 64)], sem).wait()
    for k in range(CH * FANOUT * FANOUT // IDX_TILE):
        pltpu.make_async_copy(
            f2_hbm.at[n2v.at[pl.ds(0, IDX_TILE)]],
            rn2.at[pl.ds(k * IDX_TILE, IDX_TILE)], sem).wait()


def _compute(ci, base, rs, rn1a, rn1b, rn2, hb, out_hbm):
    sixth = jnp.float32(1.0 / FANOUT)
    cb = base + ci * CH

    @pl.loop(0, CH)
    def elem(e):
        zero = jnp.zeros((16,), jnp.float32)
        acc0 = zero
        acc1 = zero
        sb0 = zero
        sb1 = zero
        for i in range(FANOUT):
            g = e * FANOUT + i
            s0 = zero
            s1 = zero
            for j in range(FANOUT):
                r = g * FANOUT + j
                s0 = s0 + rn2[r, pl.ds(0, 16)]
                s1 = s1 + rn2[r, pl.ds(4, 16)]
            q0 = jnp.maximum(rn1a[g, pl.ds(0, 16)] + sixth * s0, 0.0)
            q1 = jnp.maximum(rn1a[g, pl.ds(4, 16)] + sixth * s1, 0.0)
            acc0 = acc0 + q0
            acc1 = acc1 + q1
            sb0 = sb0 + rn1b[g, pl.ds(0, 16)]
            sb1 = sb1 + rn1b[g, pl.ds(4, 16)]
        hs0 = jnp.maximum(rs[e, pl.ds(0, 16)] + sixth * sb0, 0.0)
        hs1 = jnp.maximum(rs[e, pl.ds(4, 16)] + sixth * sb1, 0.0)
        hb[e, pl.ds(0, 16)] = hs0
        hb[e, pl.ds(4, 16)] = hs1
        hb[e, pl.ds(20, 16)] = sixth * acc0
        hb[e, pl.ds(24, 16)] = sixth * acc1

    pltpu.sync_copy(hb, out_hbm.at[pl.ds(cb, CH)])


def _gather_body(f1_hbm, f2_hbm, idx_hbm, n1_hbm, n2_hbm, out_hbm,
                 idxv, n1v, n2v,
                 rsA, rn1aA, rn1bA, rn2A,
                 rsB, rn1aB, rn1bB, rn2B,
                 hb, semA, semB):
    wid = lax.axis_index("s") * NC + lax.axis_index("c")
    base = wid * BPW
    pltpu.sync_copy(idx_hbm.at[pl.ds(base, BPW)], idxv)
    pltpu.sync_copy(n1_hbm.at[pl.ds(base * FANOUT, BPW * FANOUT)], n1v)
    pltpu.sync_copy(n2_hbm.at[pl.ds(base * FANOUT * FANOUT,
                                    BPW * FANOUT * FANOUT)], n2v)

    bufA = (rsA, rn1aA, rn1bA, rn2A)
    bufB = (rsB, rn1aB, rn1bB, rn2B)
    _fire(0, f1_hbm, f2_hbm, idxv, n1v, n2v, *bufA, semA)

    @pl.loop(0, NCHUNK // 2)
    def step(h):
        ci0 = 2 * h
        _fire(ci0 + 1, f1_hbm, f2_hbm, idxv, n1v, n2v, *bufB, semB)
        _drain(f1_hbm, f2_hbm, idxv, n1v, n2v, *bufA, semA)
        _compute(ci0, base, *bufA, hb, out_hbm)

        @pl.when(h + 1 < NCHUNK // 2)
        def _():
            _fire(ci0 + 2, f1_hbm, f2_hbm, idxv, n1v, n2v, *bufA, semA)

        _drain(f1_hbm, f2_hbm, idxv, n1v, n2v, *bufB, semB)
        _compute(ci0 + 1, base, *bufB, hb, out_hbm)


def _gather_kernel(F1, F2, idx, n1f, n2f):
    mesh = plsc.VectorSubcoreMesh(core_axis_name="c", subcore_axis_name="s")
    run = functools.partial(
        pl.kernel,
        out_type=jax.ShapeDtypeStruct((BATCH, HD), jnp.float32),
        mesh=mesh,
        compiler_params=pltpu.CompilerParams(use_tc_tiling_on_sc=False, needs_layout_passes=False),
        scratch_types=[
            pltpu.VMEM((BPW,), jnp.int32),
            pltpu.VMEM((BPW * FANOUT,), jnp.int32),
            pltpu.VMEM((BPW * FANOUT * FANOUT,), jnp.int32),
            pltpu.VMEM((CH, TD), jnp.float32),
            pltpu.VMEM((CH * FANOUT, TD), jnp.float32),
            pltpu.VMEM((CH * FANOUT, TD), jnp.float32),
            pltpu.VMEM((CH * FANOUT * FANOUT, TD), jnp.float32),
            pltpu.VMEM((CH, TD), jnp.float32),
            pltpu.VMEM((CH * FANOUT, TD), jnp.float32),
            pltpu.VMEM((CH * FANOUT, TD), jnp.float32),
            pltpu.VMEM((CH * FANOUT * FANOUT, TD), jnp.float32),
            pltpu.VMEM((CH, HD), jnp.float32),
            pltpu.SemaphoreType.DMA,
            pltpu.SemaphoreType.DMA,
        ],
    )(_gather_body)
    return run(F1, F2, idx, n1f, n2f)


def kernel(features, idx, first_order_neighs, second_order_neighs,
           W1, b1, W2, b2):
    F1, F2 = _make_tables(features, W1, b1)
    H = _gather_kernel(F1, F2, idx, first_order_neighs.reshape(-1),
                       second_order_neighs.reshape(-1))
    return _head(H, W2, b2)


# speedup vs baseline: 1.2527x; 1.0002x over previous
import functools

import jax
import jax.numpy as jnp
from jax import lax
from jax.experimental import pallas as pl
from jax.experimental.pallas import tpu as pltpu
from jax.experimental.pallas import tpu_sc as plsc

N_NODES = 100000
IN_DIM = 60
BATCH = 16384
FANOUT = 6
DIMS = 20
TD = 24
HD = 128

NC, NS = 2, 16
NW = NC * NS
BPW = BATCH // NW
CH = 32
NCHUNK = BPW // CH
IDX_TILE = 128


def _table_body(x_ref, w1a_ref, w1b_ref, b1_ref, f1_ref, f2_ref):
    x = x_ref[...]
    f1_ref[...] = (jnp.dot(x, w1a_ref[...], preferred_element_type=jnp.float32)
                   + b1_ref[...])
    f2_ref[...] = jnp.dot(x, w1b_ref[...], preferred_element_type=jnp.float32)


def _make_tables(features, W1, b1):
    rows = features.shape[0]
    blk = 8192
    grid = (rows + blk - 1) // blk
    return pl.pallas_call(
        _table_body,
        grid=(grid,),
        in_specs=[
            pl.BlockSpec((blk, IN_DIM), lambda i: (i, 0)),
            pl.BlockSpec((IN_DIM, TD), lambda i: (0, 0)),
            pl.BlockSpec((IN_DIM, TD), lambda i: (0, 0)),
            pl.BlockSpec((1, TD), lambda i: (0, 0)),
        ],
        out_specs=[
            pl.BlockSpec((blk, TD), lambda i: (i, 0)),
            pl.BlockSpec((blk, TD), lambda i: (i, 0)),
        ],
        out_shape=[
            jax.ShapeDtypeStruct((rows, TD), jnp.float32),
            jax.ShapeDtypeStruct((rows, TD), jnp.float32),
        ],
    )(features,
      jnp.pad(W1[:IN_DIM], ((0, 0), (0, TD - DIMS))),
      jnp.pad(W1[IN_DIM:], ((0, 0), (0, TD - DIMS))),
      jnp.pad(b1, (0, TD - DIMS)).reshape(1, TD))


def _head_body(h_ref, w2_ref, b2_ref, o_ref):
    h = h_ref[...][:, :2 * DIMS]
    acc = jnp.dot(h, w2_ref[...], preferred_element_type=jnp.float32)
    o_ref[...] = jnp.maximum(acc + b2_ref[...], 0.0)


def _head(H, W2, b2):
    blk = 2048
    return pl.pallas_call(
        _head_body,
        grid=(BATCH // blk,),
        in_specs=[
            pl.BlockSpec((blk, HD), lambda i: (i, 0)),

            pl.BlockSpec((2 * DIMS, DIMS), lambda i: (0, 0)),
            pl.BlockSpec((1, DIMS), lambda i: (0, 0)),
        ],
        out_specs=pl.BlockSpec((blk, DIMS), lambda i: (i, 0)),
        out_shape=jax.ShapeDtypeStruct((BATCH, DIMS), jnp.float32),
    )(H, W2, b2.reshape(1, DIMS))


N1_OFF = BATCH
N2_OFF = BATCH + BATCH * FANOUT


def _fire(ci, f1_hbm, f2_hbm, idxv, n1v, n2v, rs, rn1a, rn1b, rn2, sem):
    pltpu.async_copy(f1_hbm.at[idxv.at[pl.ds(ci * CH, CH)]], rs, sem)
    n1o = ci * CH * FANOUT
    pltpu.async_copy(f1_hbm.at[n1v.at[pl.ds(n1o, 128)]],
                     rn1a.at[pl.ds(0, 128)], sem)
    pltpu.async_copy(f1_hbm.at[n1v.at[pl.ds(n1o + 128, 64)]],
                     rn1a.at[pl.ds(128, 64)], sem)
    pltpu.async_copy(f2_hbm.at[n1v.at[pl.ds(n1o, 128)]],
                     rn1b.at[pl.ds(0, 128)], sem)
    pltpu.async_copy(f2_hbm.at[n1v.at[pl.ds(n1o + 128, 64)]],
                     rn1b.at[pl.ds(128, 64)], sem)
    n2o = ci * CH * FANOUT * FANOUT
    for k in range(CH * FANOUT * FANOUT // IDX_TILE):
        pltpu.async_copy(f2_hbm.at[n2v.at[pl.ds(n2o + k * IDX_TILE, IDX_TILE)]],
                         rn2.at[pl.ds(k * IDX_TILE, IDX_TILE)], sem)


def _drain(f1_hbm, f2_hbm, idxv, n1v, n2v, rs, rn1a, rn1b, rn2, sem):
    pltpu.make_async_copy(f1_hbm.at[idxv.at[pl.ds(0, CH)]], rs, sem).wait()
    pltpu.make_async_copy(f1_hbm.at[n1v.at[pl.ds(0, 128)]],
                          rn1a.at[pl.ds(0, 128)], sem).wait()
    pltpu.make_async_copy(f1_hbm.at[n1v.at[pl.ds(0, 64)]],
                          rn1a.at[pl.ds(128, 64)], sem).wait()
    pltpu.make_async_copy(f2_hbm.at[n1v.at[pl.ds(0, 128)]],
                          rn1b.at[pl.ds(0, 128)], sem).wait()
    pltpu.make_async_copy(f2_hbm.at[n1v.at[pl.ds(0, 64)]],
                          rn1b.at[pl.ds(128, 64)], sem).wait()
    for k in range(CH * FANOUT * FANOUT // IDX_TILE):
        pltpu.make_async_copy(
            f2_hbm.at[n2v.at[pl.ds(0, IDX_TILE)]],
            rn2.at[pl.ds(k * IDX_TILE, IDX_TILE)], sem).wait()


def _compute(ci, base, rs, rn1a, rn1b, rn2, hb, out_hbm):
    sixth = jnp.float32(1.0 / FANOUT)
    cb = base + ci * CH

    @pl.loop(0, CH, unroll=2)
    def elem(e):
        zero = jnp.zeros((16,), jnp.float32)
        acc0 = zero
        acc1 = zero
        sb0 = zero
        sb1 = zero
        for i in range(FANOUT):
            g = e * FANOUT + i
            s0 = zero
            s1 = zero
            for j in range(FANOUT):
                r = g * FANOUT + j
                s0 = s0 + rn2[r, pl.ds(0, 16)]
                s1 = s1 + rn2[r, pl.ds(4, 16)]
            q0 = jnp.maximum(rn1a[g, pl.ds(0, 16)] + sixth * s0, 0.0)
            q1 = jnp.maximum(rn1a[g, pl.ds(4, 16)] + sixth * s1, 0.0)
            acc0 = acc0 + q0
            acc1 = acc1 + q1
            sb0 = sb0 + rn1b[g, pl.ds(0, 16)]
            sb1 = sb1 + rn1b[g, pl.ds(4, 16)]
        hs0 = jnp.maximum(rs[e, pl.ds(0, 16)] + sixth * sb0, 0.0)
        hs1 = jnp.maximum(rs[e, pl.ds(4, 16)] + sixth * sb1, 0.0)
        hb[e, pl.ds(0, 16)] = hs0
        hb[e, pl.ds(4, 16)] = hs1
        hb[e, pl.ds(20, 16)] = sixth * acc0
        hb[e, pl.ds(24, 16)] = sixth * acc1

    pltpu.sync_copy(hb, out_hbm.at[pl.ds(cb, CH)])


def _gather_body(f1_hbm, f2_hbm, idx_hbm, n1_hbm, n2_hbm, out_hbm,
                 idxv, n1v, n2v,
                 rsA, rn1aA, rn1bA, rn2A,
                 rsB, rn1aB, rn1bB, rn2B,
                 hb, semA, semB):
    wid = lax.axis_index("s") * NC + lax.axis_index("c")
    base = wid * BPW
    pltpu.sync_copy(idx_hbm.at[pl.ds(base, BPW)], idxv)
    pltpu.sync_copy(n1_hbm.at[pl.ds(base * FANOUT, BPW * FANOUT)], n1v)
    pltpu.sync_copy(n2_hbm.at[pl.ds(base * FANOUT * FANOUT,
                                    BPW * FANOUT * FANOUT)], n2v)

    bufA = (rsA, rn1aA, rn1bA, rn2A)
    bufB = (rsB, rn1aB, rn1bB, rn2B)
    _fire(0, f1_hbm, f2_hbm, idxv, n1v, n2v, *bufA, semA)

    @pl.loop(0, NCHUNK // 2)
    def step(h):
        ci0 = 2 * h
        _fire(ci0 + 1, f1_hbm, f2_hbm, idxv, n1v, n2v, *bufB, semB)
        _drain(f1_hbm, f2_hbm, idxv, n1v, n2v, *bufA, semA)
        _compute(ci0, base, *bufA, hb, out_hbm)

        @pl.when(h + 1 < NCHUNK // 2)
        def _():
            _fire(ci0 + 2, f1_hbm, f2_hbm, idxv, n1v, n2v, *bufA, semA)

        _drain(f1_hbm, f2_hbm, idxv, n1v, n2v, *bufB, semB)
        _compute(ci0 + 1, base, *bufB, hb, out_hbm)


def _gather_kernel(F1, F2, idx, n1f, n2f):
    mesh = plsc.VectorSubcoreMesh(core_axis_name="c", subcore_axis_name="s")
    run = functools.partial(
        pl.kernel,
        out_type=jax.ShapeDtypeStruct((BATCH, HD), jnp.float32),
        mesh=mesh,
        compiler_params=pltpu.CompilerParams(use_tc_tiling_on_sc=False, needs_layout_passes=False),
        scratch_types=[
            pltpu.VMEM((BPW,), jnp.int32),
            pltpu.VMEM((BPW * FANOUT,), jnp.int32),
            pltpu.VMEM((BPW * FANOUT * FANOUT,), jnp.int32),
            pltpu.VMEM((CH, TD), jnp.float32),
            pltpu.VMEM((CH * FANOUT, TD), jnp.float32),
            pltpu.VMEM((CH * FANOUT, TD), jnp.float32),
            pltpu.VMEM((CH * FANOUT * FANOUT, TD), jnp.float32),
            pltpu.VMEM((CH, TD), jnp.float32),
            pltpu.VMEM((CH * FANOUT, TD), jnp.float32),
            pltpu.VMEM((CH * FANOUT, TD), jnp.float32),
            pltpu.VMEM((CH * FANOUT * FANOUT, TD), jnp.float32),
            pltpu.VMEM((CH, HD), jnp.float32),
            pltpu.SemaphoreType.DMA,
            pltpu.SemaphoreType.DMA,
        ],
    )(_gather_body)
    return run(F1, F2, idx, n1f, n2f)


def kernel(features, idx, first_order_neighs, second_order_neighs,
           W1, b1, W2, b2):
    F1, F2 = _make_tables(features, W1, b1)
    H = _gather_kernel(F1, F2, idx, first_order_neighs.reshape(-1),
                       second_order_neighs.reshape(-1))
    return _head(H, W2, b2)


# speedup vs baseline: 1.2531x; 1.0003x over previous
import functools

import jax
import jax.numpy as jnp
from jax import lax
from jax.experimental import pallas as pl
from jax.experimental.pallas import tpu as pltpu
from jax.experimental.pallas import tpu_sc as plsc

N_NODES = 100000
IN_DIM = 60
BATCH = 16384
FANOUT = 6
DIMS = 20
TD = 24
HD = 128

NC, NS = 2, 16
NW = NC * NS
BPW = BATCH // NW
CH = 32
NCHUNK = BPW // CH
IDX_TILE = 128


def _table_body(x_ref, w1a_ref, w1b_ref, b1_ref, f1_ref, f2_ref):
    x = x_ref[...]
    f1_ref[...] = (jnp.dot(x, w1a_ref[...], preferred_element_type=jnp.float32)
                   + b1_ref[...])
    f2_ref[...] = jnp.dot(x, w1b_ref[...], preferred_element_type=jnp.float32)


def _make_tables(features, W1, b1):
    rows = features.shape[0]
    blk = 8192
    grid = (rows + blk - 1) // blk
    return pl.pallas_call(
        _table_body,
        grid=(grid,),
        in_specs=[
            pl.BlockSpec((blk, IN_DIM), lambda i: (i, 0)),
            pl.BlockSpec((IN_DIM, TD), lambda i: (0, 0)),
            pl.BlockSpec((IN_DIM, TD), lambda i: (0, 0)),
            pl.BlockSpec((1, TD), lambda i: (0, 0)),
        ],
        out_specs=[
            pl.BlockSpec((blk, TD), lambda i: (i, 0)),
            pl.BlockSpec((blk, TD), lambda i: (i, 0)),
        ],
        out_shape=[
            jax.ShapeDtypeStruct((rows, TD), jnp.float32),
            jax.ShapeDtypeStruct((rows, TD), jnp.float32),
        ],
    )(features,
      jnp.pad(W1[:IN_DIM], ((0, 0), (0, TD - DIMS))),
      jnp.pad(W1[IN_DIM:], ((0, 0), (0, TD - DIMS))),
      jnp.pad(b1, (0, TD - DIMS)).reshape(1, TD))


def _head_body(h_ref, w2_ref, b2_ref, o_ref):
    h = h_ref[...][:, :2 * DIMS]
    acc = jnp.dot(h, w2_ref[...], preferred_element_type=jnp.float32)
    o_ref[...] = jnp.maximum(acc + b2_ref[...], 0.0)


def _head(H, W2, b2):
    blk = 2048
    return pl.pallas_call(
        _head_body,
        grid=(BATCH // blk,),
        in_specs=[
            pl.BlockSpec((blk, HD), lambda i: (i, 0)),

            pl.BlockSpec((2 * DIMS, DIMS), lambda i: (0, 0)),
            pl.BlockSpec((1, DIMS), lambda i: (0, 0)),
        ],
        out_specs=pl.BlockSpec((blk, DIMS), lambda i: (i, 0)),
        out_shape=jax.ShapeDtypeStruct((BATCH, DIMS), jnp.float32),
    )(H, W2, b2.reshape(1, DIMS))


def _fire(ci, f1_hbm, f2_hbm, idxv, n1v, n2v, rs, rn1a, rn1b, rn2, sem):
    pltpu.async_copy(f1_hbm.at[idxv.at[pl.ds(ci * CH, CH)]], rs, sem)
    n1o = ci * CH * FANOUT
    pltpu.async_copy(f1_hbm.at[n1v.at[pl.ds(n1o, 128)]],
                     rn1a.at[pl.ds(0, 128)], sem)
    pltpu.async_copy(f1_hbm.at[n1v.at[pl.ds(n1o + 128, 64)]],
                     rn1a.at[pl.ds(128, 64)], sem)
    pltpu.async_copy(f2_hbm.at[n1v.at[pl.ds(n1o, 128)]],
                     rn1b.at[pl.ds(0, 128)], sem)
    pltpu.async_copy(f2_hbm.at[n1v.at[pl.ds(n1o + 128, 64)]],
                     rn1b.at[pl.ds(128, 64)], sem)
    n2o = ci * CH * FANOUT * FANOUT
    for k in range(CH * FANOUT * FANOUT // IDX_TILE):
        pltpu.async_copy(f2_hbm.at[n2v.at[pl.ds(n2o + k * IDX_TILE, IDX_TILE)]],
                         rn2.at[pl.ds(k * IDX_TILE, IDX_TILE)], sem)


def _drain(f1_hbm, f2_hbm, idxv, n1v, n2v, rs, rn1a, rn1b, rn2, sem):
    pltpu.make_async_copy(f1_hbm.at[idxv.at[pl.ds(0, CH)]], rs, sem).wait()
    pltpu.make_async_copy(f1_hbm.at[n1v.at[pl.ds(0, 128)]],
                          rn1a.at[pl.ds(0, 128)], sem).wait()
    pltpu.make_async_copy(f1_hbm.at[n1v.at[pl.ds(0, 64)]],
                          rn1a.at[pl.ds(128, 64)], sem).wait()
    pltpu.make_async_copy(f2_hbm.at[n1v.at[pl.ds(0, 128)]],
                          rn1b.at[pl.ds(0, 128)], sem).wait()
    pltpu.make_async_copy(f2_hbm.at[n1v.at[pl.ds(0, 64)]],
                          rn1b.at[pl.ds(128, 64)], sem).wait()
    for k in range(CH * FANOUT * FANOUT // IDX_TILE):
        pltpu.make_async_copy(
            f2_hbm.at[n2v.at[pl.ds(0, IDX_TILE)]],
            rn2.at[pl.ds(k * IDX_TILE, IDX_TILE)], sem).wait()


def _compute(ci, base, rs, rn1a, rn1b, rn2, hb, out_hbm):
    sixth = jnp.float32(1.0 / FANOUT)
    cb = base + ci * CH

    @pl.loop(0, CH)
    def elem(e):
        zero = jnp.zeros((16,), jnp.float32)
        acc0 = zero
        acc1 = zero
        sb0 = zero
        sb1 = zero
        for i in range(FANOUT):
            g = e * FANOUT + i
            s0 = zero
            s1 = zero
            for j in range(FANOUT):
                r = g * FANOUT + j
                s0 = s0 + rn2[r, pl.ds(0, 16)]
                s1 = s1 + rn2[r, pl.ds(4, 16)]
            q0 = jnp.maximum(rn1a[g, pl.ds(0, 16)] + sixth * s0, 0.0)
            q1 = jnp.maximum(rn1a[g, pl.ds(4, 16)] + sixth * s1, 0.0)
            acc0 = acc0 + q0
            acc1 = acc1 + q1
            sb0 = sb0 + rn1b[g, pl.ds(0, 16)]
            sb1 = sb1 + rn1b[g, pl.ds(4, 16)]
        hs0 = jnp.maximum(rs[e, pl.ds(0, 16)] + sixth * sb0, 0.0)
        hs1 = jnp.maximum(rs[e, pl.ds(4, 16)] + sixth * sb1, 0.0)
        hb[e, pl.ds(0, 16)] = hs0
        hb[e, pl.ds(4, 16)] = hs1
        hb[e, pl.ds(20, 16)] = sixth * acc0
        hb[e, pl.ds(24, 16)] = sixth * acc1

    pltpu.sync_copy(hb, out_hbm.at[pl.ds(cb, CH)])


def _gather_body(f1_hbm, f2_hbm, idx_hbm, n1_hbm, n2_hbm, out_hbm,
                 idxv, n1v, n2v,
                 rsA, rn1aA, rn1bA, rn2A,
                 rsB, rn1aB, rn1bB, rn2B,
                 hb, semA, semB):
    wid = lax.axis_index("s") * NC + lax.axis_index("c")
    base = wid * BPW
    pltpu.sync_copy(idx_hbm.at[pl.ds(base, BPW)], idxv)
    pltpu.sync_copy(n1_hbm.at[pl.ds(base * FANOUT, BPW * FANOUT)], n1v)
    pltpu.sync_copy(n2_hbm.at[pl.ds(base * FANOUT * FANOUT,
                                    BPW * FANOUT * FANOUT)], n2v)

    bufA = (rsA, rn1aA, rn1bA, rn2A)
    bufB = (rsB, rn1aB, rn1bB, rn2B)
    _fire(0, f1_hbm, f2_hbm, idxv, n1v, n2v, *bufA, semA)

    @pl.loop(0, NCHUNK // 2)
    def step(h):
        ci0 = 2 * h
        _fire(ci0 + 1, f1_hbm, f2_hbm, idxv, n1v, n2v, *bufB, semB)
        _drain(f1_hbm, f2_hbm, idxv, n1v, n2v, *bufA, semA)
        _compute(ci0, base, *bufA, hb, out_hbm)

        @pl.when(h + 1 < NCHUNK // 2)
        def _():
            _fire(ci0 + 2, f1_hbm, f2_hbm, idxv, n1v, n2v, *bufA, semA)

        _drain(f1_hbm, f2_hbm, idxv, n1v, n2v, *bufB, semB)
        _compute(ci0 + 1, base, *bufB, hb, out_hbm)


def _gather_kernel(F1, F2, idx, n1f, n2f):
    mesh = plsc.VectorSubcoreMesh(core_axis_name="c", subcore_axis_name="s")
    run = functools.partial(
        pl.kernel,
        out_type=jax.ShapeDtypeStruct((BATCH, HD), jnp.float32),
        mesh=mesh,
        compiler_params=pltpu.CompilerParams(use_tc_tiling_on_sc=False, needs_layout_passes=False),
        scratch_types=[
            pltpu.VMEM((BPW,), jnp.int32),
            pltpu.VMEM((BPW * FANOUT,), jnp.int32),
            pltpu.VMEM((BPW * FANOUT * FANOUT,), jnp.int32),
            pltpu.VMEM((CH, TD), jnp.float32),
            pltpu.VMEM((CH * FANOUT, TD), jnp.float32),
            pltpu.VMEM((CH * FANOUT, TD), jnp.float32),
            pltpu.VMEM((CH * FANOUT * FANOUT, TD), jnp.float32),
            pltpu.VMEM((CH, TD), jnp.float32),
            pltpu.VMEM((CH * FANOUT, TD), jnp.float32),
            pltpu.VMEM((CH * FANOUT, TD), jnp.float32),
            pltpu.VMEM((CH * FANOUT * FANOUT, TD), jnp.float32),
            pltpu.VMEM((CH, HD), jnp.float32),
            pltpu.SemaphoreType.DMA,
            pltpu.SemaphoreType.DMA,
        ],
    )(_gather_body)
    return run(F1, F2, idx, n1f, n2f)


def kernel(features, idx, first_order_neighs, second_order_neighs,
           W1, b1, W2, b2):
    F1, F2 = _make_tables(features, W1, b1)
    H = _gather_kernel(F1, F2, idx, first_order_neighs.reshape(-1),
                       second_order_neighs.reshape(-1))
    return _head(H, W2, b2)
